# baseline (speedup 1.0000x reference)
import functools

import jax
import jax.numpy as jnp
from jax import lax
from jax.experimental import pallas as pl
from jax.experimental.pallas import tpu as pltpu
from jax.experimental.pallas import tpu_sc as plsc

_N = 10000
_NP = 10240
_DUM = _N
_E = 320000
_EPF = 344064
_EPC = 327680
_K = 128
_RPT = _NP // 16
_H = 128
_NB = 2048
_NG = _NP // _NB

_mesh = plsc.VectorSubcoreMesh(core_axis_name="c", subcore_axis_name="s")


def _fill_identity(idqr):
  for g in range(5):
    idqr[pl.ds(g * 16, 16)] = lax.iota(jnp.int32, 16) + g * 16


def _zero_rows(buf, nrows):
  zv = jnp.zeros((16,), jnp.float32)

  def _zrow(i, _):
    for jj in range(8):
      buf[i, pl.ds(jj * 16, 16)] = zv
    return 0

  lax.fori_loop(0, nrows, _zrow, 0)


def _zero_shared_slice(wch, acc, s):
  _zero_rows(wch, 128)
  for ch in range(5):
    pltpu.sync_copy(wch, acc.at[pl.ds(s * _RPT + ch * 128, 128)])


def _att_sc(a_s, a_d, srcF, dstF, exo, deno,
            asb, adb, denla, denlb, idqr, sslab, dslab, exba, exbb,
            den_sha, den_shb):
  c = lax.axis_index("c")
  s = lax.axis_index("s")
  ept = _EPF // 16
  _fill_identity(idqr)
  ha = c * 2
  pltpu.sync_copy(a_s.at[pl.ds(ha * _NP, _NP)], asb.at[pl.ds(0, _NP)])
  pltpu.sync_copy(a_s.at[pl.ds((ha + 1) * _NP, _NP)], asb.at[pl.ds(_NP, _NP)])
  pltpu.sync_copy(a_d.at[pl.ds(ha * _NP, _NP)], adb.at[pl.ds(0, _NP)])
  pltpu.sync_copy(a_d.at[pl.ds((ha + 1) * _NP, _NP)], adb.at[pl.ds(_NP, _NP)])
  _zero_rows(denla, 80)
  _zero_rows(denlb, 80)

  @pl.when(s < 10)
  def _():
    pltpu.sync_copy(denla.at[pl.ds(0, 8)], den_sha.at[pl.ds(s * 8, 8)])
    pltpu.sync_copy(denla.at[pl.ds(0, 8)], den_shb.at[pl.ds(s * 8, 8)])

  plsc.subcore_barrier()

  def _slab(sl, _):
    e0 = s * ept + sl * 1024
    pltpu.sync_copy(srcF.at[pl.ds(e0, 1024)], sslab)
    pltpu.sync_copy(dstF.at[pl.ds(e0, 1024)], dslab)
    for b in range(8):
      for g in range(8):
        o = pl.ds(b * 128 + g * 16, 16)
        s16 = sslab[o]
        d16 = dslab[o]
        dr = lax.shift_right_logical(d16, 7)
        dc = lax.bitwise_and(d16, 127)
        asva = plsc.load_gather(asb, [s16])
        adva = plsc.load_gather(adb, [d16])
        ala = asva + adva
        exa = jnp.exp(jnp.maximum(ala, 0.2 * ala))
        exba[o] = exa
        plsc.addupdate_scatter(denla, [dr, dc], exa)
        asvb = plsc.load_gather(asb, [s16 + _NP])
        advb = plsc.load_gather(adb, [d16 + _NP])
        alb = asvb + advb
        exb_ = jnp.exp(jnp.maximum(alb, 0.2 * alb))
        exbb[o] = exb_
        plsc.addupdate_scatter(denlb, [dr, dc], exb_)
    pltpu.sync_copy(exba, exo.at[pl.ds(ha * _EPF + e0, 1024)])
    pltpu.sync_copy(exbb, exo.at[pl.ds((ha + 1) * _EPF + e0, 1024)])
    return 0

  lax.fori_loop(0, ept // 1024, _slab, 0)
  pltpu.sync_copy(denla, den_sha.at[idqr], add=True)
  pltpu.sync_copy(denlb, den_shb.at[idqr], add=True)
  plsc.subcore_barrier()

  @pl.when(s < 10)
  def _():
    pltpu.sync_copy(den_sha.at[pl.ds(s * 8, 8)],
                    deno.at[pl.ds(ha * 80 + s * 8, 8)])
    pltpu.sync_copy(den_shb.at[pl.ds(s * 8, 8)],
                    deno.at[pl.ds((ha + 1) * 80 + s * 8, 8)])


def _gat_sc(hh, exo, deno, b1f, srcF, dstF, h1o,
            denl, sslab, dslab, exslab, srcba, srcbb, dstba, dstbb,
            rowsa, rowsb, b1b, acc, sema, semb, sems):
  c = lax.axis_index("c")
  s = lax.axis_index("s")
  ept = _EPF // 16
  for j in range(2):
    head = c * 2 + j
    hoff = head * _NP
    pltpu.sync_copy(b1f.at[pl.ds(head * _H, _H)], b1b)
    _zero_rows(rowsa, 128)
    for ch in range(5):
      pltpu.sync_copy(rowsa, acc.at[pl.ds(s * _RPT + ch * 128, 128)])
    plsc.subcore_barrier()

    def _slab(sl, _):
      e0 = s * ept + sl * 1024
      pltpu.sync_copy(srcF.at[pl.ds(e0, 1024)], sslab)
      pltpu.sync_copy(dstF.at[pl.ds(e0, 1024)], dslab)
      pltpu.sync_copy(exo.at[pl.ds(head * _EPF + e0, 1024)], exslab)
      def _bat(b, _):
        for g in range(8):
          o = pl.ds(b * 128 + g * 16, 16)
          gsl = pl.ds(g * 16, 16)
          srcba[gsl] = sslab[o] + hoff
          dstba[gsl] = dslab[o]
        pltpu.async_copy(hh.at[srcba], rowsa, sema).wait()

        def _scale(k, _):
          wv = plsc.load_gather(
              exslab, [jnp.full((16,), b * 128 + k, jnp.int32)])
          for jj in range(8):
            sl2 = pl.ds(jj * 16, 16)
            rowsa[k, sl2] = rowsa[k, sl2] * wv
          return 0

        lax.fori_loop(0, 128, _scale, 0)
        pltpu.sync_copy(rowsa, acc.at[dstba], add=True)
        return 0

      lax.fori_loop(0, 8, _bat, 0)
      return 0

    lax.fori_loop(0, ept // 1024, _slab, 0)
    plsc.subcore_barrier()
    pltpu.sync_copy(deno.at[pl.ds(head * 80, 80)], denl)
    for ch in range(5):
      r0 = s * _RPT + ch * 128
      pltpu.sync_copy(acc.at[pl.ds(r0, 128)], rowsa)

      def _wrow(i, _):
        f = r0 + i
        dsp = plsc.load_gather(
            denl, [jnp.full((16,), lax.shift_right_logical(f, 7), jnp.int32),
                   jnp.full((16,), lax.bitwise_and(f, 127), jnp.int32)]
        ) + 1e-16
        for jj in range(8):
          sl2 = pl.ds(jj * 16, 16)
          rowsa[i, sl2] = jnp.maximum(rowsa[i, sl2] / dsp + b1b[sl2], 0.0)
        return 0

      lax.fori_loop(0, 128, _wrow, 0)
      pltpu.sync_copy(rowsa, h1o.at[pl.ds(hoff + r0, 128)])
    plsc.subcore_barrier()


def _sage_sc(h1f, srcE, dstE, meanf, cnto,
             cntl, idqr, sslab, dslab, srcba, srcbb, dstba, dstbb,
             rowsa, rowsb, acc, cnt_sh, sema, semb, sems):
  c = lax.axis_index("c")
  s = lax.axis_index("s")
  ept = _EPC // 16
  _fill_identity(idqr)
  ones = jnp.full((16,), 1.0, jnp.float32)
  _zero_rows(cntl, 80)

  @pl.when(s < 10)
  def _():
    pltpu.sync_copy(cntl.at[pl.ds(0, 8)], cnt_sh.at[pl.ds(s * 8, 8)])
  for j in range(2):
    chunk = c * 2 + j
    coff = chunk * _NP
    _zero_shared_slice(rowsa, acc, s)
    plsc.subcore_barrier()

    def _slab(sl, _):
      e0 = s * ept + sl * 1024
      pltpu.sync_copy(srcE.at[pl.ds(e0, 1024)], sslab)
      pltpu.sync_copy(dstE.at[pl.ds(e0, 1024)], dslab)
      def _bat(b, _):
        for g in range(8):
          o = pl.ds(b * 128 + g * 16, 16)
          gsl = pl.ds(g * 16, 16)
          s16 = sslab[o]
          srcba[gsl] = s16 + coff
          d16 = dslab[o]
          dstba[gsl] = d16
          if j == 0:
            plsc.addupdate_scatter(
                cntl,
                [lax.shift_right_logical(d16, 7),
                 lax.bitwise_and(d16, 127)], ones)
        pltpu.async_copy(h1f.at[srcba], rowsa, sema).wait()
        pltpu.sync_copy(rowsa, acc.at[dstba], add=True)
        return 0

      lax.fori_loop(0, 8, _bat, 0)
      return 0

    lax.fori_loop(0, ept // 1024, _slab, 0)
    if j == 0:
      pltpu.sync_copy(cntl, cnt_sh.at[idqr], add=True)
    plsc.subcore_barrier()
    if j == 0:
      pltpu.sync_copy(cnt_sh, cntl)

      @pl.when(jnp.logical_and(c == 0, s < 10))
      def _():
        pltpu.sync_copy(cntl.at[pl.ds(s * 8, 8)], cnto.at[pl.ds(s * 8, 8)])
    for ch in range(5):
      r0 = s * _RPT + ch * 128
      pltpu.sync_copy(acc.at[pl.ds(r0, 128)], rowsa)

      def _wrow(i, _):
        f = r0 + i
        csp = jnp.maximum(
            plsc.load_gather(
                cntl,
                [jnp.full((16,), lax.shift_right_logical(f, 7), jnp.int32),
                 jnp.full((16,), lax.bitwise_and(f, 127), jnp.int32)]), 1.0)
        for jj in range(8):
          sl2 = pl.ds(jj * 16, 16)
          rowsa[i, sl2] = rowsa[i, sl2] / csp
        return 0

      lax.fori_loop(0, 128, _wrow, 0)
      pltpu.sync_copy(rowsa, meanf.at[pl.ds(coff + r0, 128)])
    plsc.subcore_barrier()


def _gcn_sc(gf, srcF, dstF, partf, srcb, dstb, rows, acc, sem):
  wch = rows
  c = lax.axis_index("c")
  s = lax.axis_index("s")
  ept = _EPF // 32
  _zero_shared_slice(wch, acc, s)
  plsc.subcore_barrier()
  t = c * 16 + s

  def _batch(b, _):
    e0 = t * ept + b * _K
    pltpu.sync_copy(srcF.at[pl.ds(e0, _K)], srcb)
    pltpu.sync_copy(dstF.at[pl.ds(e0, _K)], dstb)
    pltpu.async_copy(gf.at[srcb], rows, sem).wait()
    pltpu.sync_copy(rows, acc.at[dstb], add=True)
    return 0

  lax.fori_loop(0, ept // _K, _batch, 0)
  plsc.subcore_barrier()
  for ch in range(5):
    r0 = s * _RPT + ch * 128
    pltpu.sync_copy(acc.at[pl.ds(r0, 128)], wch)
    pltpu.sync_copy(wch, partf.at[pl.ds(c * _NP + r0, 128)])


def _tc1_body(x_r, w1_r, asr_r, adr_r, hh_r, aso_r, ado_r):
  hk = jnp.dot(x_r[...], w1_r[0], preferred_element_type=jnp.float32)
  hh_r[0] = hk
  aso_r[0, 0] = jnp.sum(hk * asr_r[0, 0][None, :], axis=1)
  ado_r[0, 0] = jnp.sum(hk * adr_r[0, 0][None, :], axis=1)


def _tc2_body(mean_r, h1_r, wl_r, wr_r, bl_r, cnt_r, w3_r, g_r, dinv_r):
  acc = jnp.broadcast_to(bl_r[...][None, :], (_NB, _H)).astype(jnp.float32)
  for k in range(4):
    acc = acc + jnp.dot(mean_r[k], wl_r[k], preferred_element_type=jnp.float32)
    acc = acc + jnp.dot(h1_r[k], wr_r[k], preferred_element_type=jnp.float32)
  h2 = jnp.maximum(acc, 0.0)
  dv = lax.rsqrt(cnt_r[...] + 1.0)
  g_r[...] = jnp.dot(h2, w3_r[...], preferred_element_type=jnp.float32) * dv
  dinv_r[...] = dv


def _tc3_body(p_r, dinv_r, b3_r, wc1_r, bc1_r, wc2_r, bc2_r, o_r):
  h3 = (p_r[0] + p_r[1]) * dinv_r[...] + b3_r[...][None, :]
  h4 = jnp.maximum(
      jnp.dot(h3, wc1_r[...], preferred_element_type=jnp.float32)
      + bc1_r[...][None, :], 0.0)
  o_r[...] = (jnp.dot(h4, wc2_r[...], preferred_element_type=jnp.float32)
              + bc2_r[...][None, :])


def kernel(x, edge_index, W1, att_src, att_dst, b1, Wl, bl, Wr, W3, b3,
           Wc1, bc1, Wc2, bc2):
  f32 = jnp.float32
  ei = edge_index.astype(jnp.int32)
  loop = jnp.arange(_N, dtype=jnp.int32)
  padF = _EPF - (_E + _N)
  padC = _EPC - _E
  srcF = jnp.concatenate([ei[0], loop, jnp.zeros((padF,), jnp.int32)])
  dstF = jnp.concatenate([ei[1], loop, jnp.full((padF,), _DUM, jnp.int32)])
  srcE = jnp.concatenate([ei[0], jnp.zeros((padC,), jnp.int32)])
  dstE = jnp.concatenate([ei[1], jnp.full((padC,), _DUM, jnp.int32)])
  xp = jnp.pad(x, ((0, _NP - _N), (0, 0)))
  W1r = W1.reshape(_H, 4, _H).transpose(1, 0, 2)
  Wlr = Wl.reshape(4, _H, _H)
  Wrr = Wr.reshape(4, _H, _H)

  hh, a_s3, a_d3 = pl.pallas_call(
      _tc1_body,
      grid=(4,),
      in_specs=[
          pl.BlockSpec((_NP, _H), lambda k: (0, 0)),
          pl.BlockSpec((1, _H, _H), lambda k: (k, 0, 0)),
          pl.BlockSpec((1, 1, _H), lambda k: (k, 0, 0)),
          pl.BlockSpec((1, 1, _H), lambda k: (k, 0, 0)),
      ],
      out_specs=[
          pl.BlockSpec((1, _NP, _H), lambda k: (k, 0, 0)),
          pl.BlockSpec((1, 1, _NP), lambda k: (k, 0, 0)),
          pl.BlockSpec((1, 1, _NP), lambda k: (k, 0, 0)),
      ],
      out_shape=[
          jax.ShapeDtypeStruct((4, _NP, _H), f32),
          jax.ShapeDtypeStruct((4, 1, _NP), f32),
          jax.ShapeDtypeStruct((4, 1, _NP), f32),
      ],
  )(xp, W1r, att_src.reshape(4, 1, _H), att_dst.reshape(4, 1, _H))

  hhf = hh.reshape(4 * _NP, _H)
  asf = a_s3.reshape(4 * _NP)
  adf = a_d3.reshape(4 * _NP)

  att = functools.partial(
      pl.kernel,
      out_type=(jax.ShapeDtypeStruct((4 * _EPF,), f32),
                jax.ShapeDtypeStruct((320, _H), f32)),
      mesh=_mesh,
      compiler_params=pltpu.CompilerParams(needs_layout_passes=False),
      scratch_types=[
          pltpu.VMEM((2 * _NP,), f32),
          pltpu.VMEM((2 * _NP,), f32),
          pltpu.VMEM((80, _H), f32),
          pltpu.VMEM((80, _H), f32),
          pltpu.VMEM((80,), jnp.int32),
          pltpu.VMEM((1024,), jnp.int32),
          pltpu.VMEM((1024,), jnp.int32),
          pltpu.VMEM((1024,), f32),
          pltpu.VMEM((1024,), f32),
          pltpu.VMEM_SHARED((80, _H), f32),
          pltpu.VMEM_SHARED((80, _H), f32),
      ],
  )(_att_sc)
  exo, deno = att(asf, adf, srcF, dstF)

  gat = functools.partial(
      pl.kernel,
      out_type=jax.ShapeDtypeStruct((4 * _NP, _H), f32),
      mesh=_mesh,
      compiler_params=pltpu.CompilerParams(needs_layout_passes=False),
      scratch_types=[
          pltpu.VMEM((80, _H), f32),
          pltpu.VMEM((1024,), jnp.int32),
          pltpu.VMEM((1024,), jnp.int32),
          pltpu.VMEM((1024,), f32),
          pltpu.VMEM((_K,), jnp.int32),
          pltpu.VMEM((_K,), jnp.int32),
          pltpu.VMEM((_K,), jnp.int32),
          pltpu.VMEM((_K,), jnp.int32),
          pltpu.VMEM((_K, _H), f32),
          pltpu.VMEM((_K, _H), f32),
          pltpu.VMEM((_H,), f32),
          pltpu.VMEM_SHARED((_NP, _H), f32),
          pltpu.SemaphoreType.DMA,
          pltpu.SemaphoreType.DMA,
          pltpu.SemaphoreType.DMA,
      ],
  )(_gat_sc)
  h1f = gat(hhf, exo, deno, b1, srcF, dstF)

  sage = functools.partial(
      pl.kernel,
      out_type=(jax.ShapeDtypeStruct((4 * _NP, _H), f32),
                jax.ShapeDtypeStruct((80, _H), f32)),
      mesh=_mesh,
      compiler_params=pltpu.CompilerParams(needs_layout_passes=False),
      scratch_types=[
          pltpu.VMEM((80, _H), f32),
          pltpu.VMEM((80,), jnp.int32),
          pltpu.VMEM((1024,), jnp.int32),
          pltpu.VMEM((1024,), jnp.int32),
          pltpu.VMEM((_K,), jnp.int32),
          pltpu.VMEM((_K,), jnp.int32),
          pltpu.VMEM((_K,), jnp.int32),
          pltpu.VMEM((_K,), jnp.int32),
          pltpu.VMEM((_K, _H), f32),
          pltpu.VMEM((_K, _H), f32),
          pltpu.VMEM_SHARED((_NP, _H), f32),
          pltpu.VMEM_SHARED((80, _H), f32),
          pltpu.SemaphoreType.DMA,
          pltpu.SemaphoreType.DMA,
          pltpu.SemaphoreType.DMA,
      ],
  )(_sage_sc)
  meanf, cnt2 = sage(h1f, srcE, dstE)
  cnt = cnt2.reshape(_NP, 1)

  g, dinv = pl.pallas_call(
      _tc2_body,
      grid=(_NG,),
      in_specs=[
          pl.BlockSpec((4, _NB, _H), lambda i: (0, i, 0)),
          pl.BlockSpec((4, _NB, _H), lambda i: (0, i, 0)),
          pl.BlockSpec((4, _H, _H), lambda i: (0, 0, 0)),
          pl.BlockSpec((4, _H, _H), lambda i: (0, 0, 0)),
          pl.BlockSpec((_H,), lambda i: (0,)),
          pl.BlockSpec((_NB, 1), lambda i: (i, 0)),
          pl.BlockSpec((_H, _H), lambda i: (0, 0)),
      ],
      out_specs=[
          pl.BlockSpec((_NB, _H), lambda i: (i, 0)),
          pl.BlockSpec((_NB, 1), lambda i: (i, 0)),
      ],
      out_shape=[
          jax.ShapeDtypeStruct((_NP, _H), f32),
          jax.ShapeDtypeStruct((_NP, 1), f32),
      ],
  )(meanf.reshape(4, _NP, _H), h1f.reshape(4, _NP, _H), Wlr, Wrr, bl, cnt, W3)

  gcn = functools.partial(
      pl.kernel,
      out_type=jax.ShapeDtypeStruct((2 * _NP, _H), f32),
      mesh=_mesh,
      compiler_params=pltpu.CompilerParams(needs_layout_passes=False),
      scratch_types=[
          pltpu.VMEM((_K,), jnp.int32),
          pltpu.VMEM((_K,), jnp.int32),
          pltpu.VMEM((_K, _H), f32),
          pltpu.VMEM_SHARED((_NP, _H), f32),
          pltpu.SemaphoreType.DMA,
      ],
  )(_gcn_sc)
  partf = gcn(g, srcF, dstF)

  out = pl.pallas_call(
      _tc3_body,
      grid=(_NG,),
      in_specs=[
          pl.BlockSpec((2, _NB, _H), lambda i: (0, i, 0)),
          pl.BlockSpec((_NB, 1), lambda i: (i, 0)),
          pl.BlockSpec((_H,), lambda i: (0,)),
          pl.BlockSpec((_H, 64), lambda i: (0, 0)),
          pl.BlockSpec((64,), lambda i: (0,)),
          pl.BlockSpec((64, 16), lambda i: (0, 0)),
          pl.BlockSpec((16,), lambda i: (0,)),
      ],
      out_specs=pl.BlockSpec((_NB, 16), lambda i: (i, 0)),
      out_shape=jax.ShapeDtypeStruct((_NP, 16), f32),
  )(partf.reshape(2, _NP, _H), dinv, b3, Wc1, bc1, Wc2, bc2)

  return out[:_N]


# speedup vs baseline: 1.6569x; 1.6569x over previous
import functools

import jax
import jax.numpy as jnp
from jax import lax
from jax.experimental import pallas as pl
from jax.experimental.pallas import tpu as pltpu
from jax.experimental.pallas import tpu_sc as plsc

_N = 10000
_NP = 10240
_DUM = _N
_E = 320000
_EPF = 331776
_EPC = 321536
_K = 128
_RPT = _NP // 16
_H = 128
_NB = 2048
_NG = _NP // _NB

_mesh = plsc.VectorSubcoreMesh(core_axis_name="c", subcore_axis_name="s")


def _fill_identity(idqr):
  for g in range(5):
    idqr[pl.ds(g * 16, 16)] = lax.iota(jnp.int32, 16) + g * 16


def _zero_rows(buf, nrows):
  zv = jnp.zeros((16,), jnp.float32)

  def _zrow(i, _):
    for jj in range(8):
      buf[i, pl.ds(jj * 16, 16)] = zv
    return 0

  lax.fori_loop(0, nrows, _zrow, 0)


def _zero_shared_slice(wch, acc, s):
  _zero_rows(wch, 128)
  for ch in range(5):
    pltpu.sync_copy(wch, acc.at[pl.ds(s * _RPT + ch * 128, 128)])


_KG = 96


def _gat_sc(hh, a_s, a_d, b1f, srcF, dstF, h1o,
            asb, adb, denl, idqr, srcb, dstb, exb, rows, b1b,
            acc, den_sh, sem):
  c = lax.axis_index("c")
  s = lax.axis_index("s")
  ept = _EPF // 16
  _fill_identity(idqr)
  for j in range(2):
    head = c * 2 + j
    hoff = head * _NP
    pltpu.sync_copy(a_s.at[pl.ds(hoff, _NP)], asb)
    pltpu.sync_copy(a_d.at[pl.ds(hoff, _NP)], adb)
    pltpu.sync_copy(b1f.at[pl.ds(head * _H, _H)], b1b)

    _zero_rows(denl, 80)
    _zero_rows(rows, 64)
    for ch in range(10):
      pltpu.sync_copy(rows.at[pl.ds(0, 64)],
                      acc.at[pl.ds(s * _RPT + ch * 64, 64)])

    @pl.when(s < 10)
    def _():
      pltpu.sync_copy(denl.at[pl.ds(0, 8)], den_sh.at[pl.ds(s * 8, 8)])

    plsc.subcore_barrier()

    def _batch(b, _):
      e0 = s * ept + b * _KG
      pltpu.sync_copy(srcF.at[pl.ds(e0, _KG)], srcb)
      pltpu.sync_copy(dstF.at[pl.ds(e0, _KG)], dstb)
      for g in range(6):
        sl = pl.ds(g * 16, 16)
        s16 = srcb[sl]
        d16 = dstb[sl]
        asv = plsc.load_gather(asb, [s16])
        adv = plsc.load_gather(adb, [d16])
        al = asv + adv
        al = jnp.maximum(al, 0.2 * al)
        ex = jnp.exp(al)
        exb[sl] = ex
        plsc.addupdate_scatter(
            denl,
            [lax.shift_right_logical(d16, 7), lax.bitwise_and(d16, 127)], ex)
        srcb[sl] = s16 + hoff
      pltpu.async_copy(hh.at[srcb], rows, sem).wait()

      def _scale(k, _):
        wv = plsc.load_gather(exb, [jnp.full((16,), k, jnp.int32)])
        for jj in range(8):
          sl2 = pl.ds(jj * 16, 16)
          rows[k, sl2] = rows[k, sl2] * wv
        return 0

      lax.fori_loop(0, _KG, _scale, 0)
      pltpu.sync_copy(rows, acc.at[dstb], add=True)
      return 0

    lax.fori_loop(0, ept // _KG, _batch, 0)
    pltpu.sync_copy(denl, den_sh.at[idqr], add=True)
    plsc.subcore_barrier()
    pltpu.sync_copy(den_sh, denl)
    for ch in range(10):
      r0 = s * _RPT + ch * 64
      pltpu.sync_copy(acc.at[pl.ds(r0, 64)], rows.at[pl.ds(0, 64)])

      def _wrow(i, _):
        f = r0 + i
        dsp = plsc.load_gather(
            denl, [jnp.full((16,), lax.shift_right_logical(f, 7), jnp.int32),
                   jnp.full((16,), lax.bitwise_and(f, 127), jnp.int32)]
        ) + 1e-16
        for jj in range(8):
          sl2 = pl.ds(jj * 16, 16)
          rows[i, sl2] = jnp.maximum(rows[i, sl2] / dsp + b1b[sl2], 0.0)
        return 0

      lax.fori_loop(0, 64, _wrow, 0)
      pltpu.sync_copy(rows.at[pl.ds(0, 64)], h1o.at[pl.ds(hoff + r0, 64)])
    plsc.subcore_barrier()


def _sage_sc(h1f, srcE, dstE, meanf, cnto,
             cntl, idqr, srcb, dstb, rows, acc, cnt_sh, sem):
  c = lax.axis_index("c")
  s = lax.axis_index("s")
  ept = _EPC // 16
  _fill_identity(idqr)
  ones = jnp.full((16,), 1.0, jnp.float32)
  _zero_rows(cntl, 80)

  @pl.when(s < 10)
  def _():
    pltpu.sync_copy(cntl.at[pl.ds(0, 8)], cnt_sh.at[pl.ds(s * 8, 8)])
  for j in range(2):
    chunk = c * 2 + j
    coff = chunk * _NP
    _zero_shared_slice(rows, acc, s)
    plsc.subcore_barrier()

    def _batch(b, _):
      e0 = s * ept + b * _K
      pltpu.sync_copy(srcE.at[pl.ds(e0, _K)], srcb)
      pltpu.sync_copy(dstE.at[pl.ds(e0, _K)], dstb)
      for g in range(8):
        sl = pl.ds(g * 16, 16)
        s16 = srcb[sl]
        if j == 0:
          d16 = dstb[sl]
          plsc.addupdate_scatter(
              cntl,
              [lax.shift_right_logical(d16, 7), lax.bitwise_and(d16, 127)],
              ones)
        srcb[sl] = s16 + coff
      pltpu.async_copy(h1f.at[srcb], rows, sem).wait()
      pltpu.sync_copy(rows, acc.at[dstb], add=True)
      return 0

    lax.fori_loop(0, ept // _K, _batch, 0)
    if j == 0:
      pltpu.sync_copy(cntl, cnt_sh.at[idqr], add=True)
    plsc.subcore_barrier()
    if j == 0:
      pltpu.sync_copy(cnt_sh, cntl)

      @pl.when(jnp.logical_and(c == 0, s < 10))
      def _():
        pltpu.sync_copy(cntl.at[pl.ds(s * 8, 8)], cnto.at[pl.ds(s * 8, 8)])
    for ch in range(5):
      r0 = s * _RPT + ch * 128
      pltpu.sync_copy(acc.at[pl.ds(r0, 128)], rows)

      def _wrow(i, _):
        f = r0 + i
        csp = jnp.maximum(
            plsc.load_gather(
                cntl,
                [jnp.full((16,), lax.shift_right_logical(f, 7), jnp.int32),
                 jnp.full((16,), lax.bitwise_and(f, 127), jnp.int32)]), 1.0)
        for jj in range(8):
          sl2 = pl.ds(jj * 16, 16)
          rows[i, sl2] = rows[i, sl2] / csp
        return 0

      lax.fori_loop(0, 128, _wrow, 0)
      pltpu.sync_copy(rows, meanf.at[pl.ds(coff + r0, 128)])
    plsc.subcore_barrier()


def _gcn_sc(gf, srcF, dstF, partf, srcb, dstb, rows, acc, sem):
  wch = rows
  c = lax.axis_index("c")
  s = lax.axis_index("s")
  ept = _EPF // 32
  _zero_shared_slice(wch, acc, s)
  plsc.subcore_barrier()
  t = c * 16 + s

  def _batch(b, _):
    e0 = t * ept + b * _K
    pltpu.sync_copy(srcF.at[pl.ds(e0, _K)], srcb)
    pltpu.sync_copy(dstF.at[pl.ds(e0, _K)], dstb)
    pltpu.async_copy(gf.at[srcb], rows, sem).wait()
    pltpu.sync_copy(rows, acc.at[dstb], add=True)
    return 0

  lax.fori_loop(0, ept // _K, _batch, 0)
  plsc.subcore_barrier()
  for ch in range(5):
    r0 = s * _RPT + ch * 128
    pltpu.sync_copy(acc.at[pl.ds(r0, 128)], wch)
    pltpu.sync_copy(wch, partf.at[pl.ds(c * _NP + r0, 128)])


def _tc1_body(x_r, w1_r, asr_r, adr_r, hh_r, aso_r, ado_r):
  hk = jnp.dot(x_r[...], w1_r[0], preferred_element_type=jnp.float32)
  hh_r[0] = hk
  aso_r[0, 0] = jnp.sum(hk * asr_r[0, 0][None, :], axis=1)
  ado_r[0, 0] = jnp.sum(hk * adr_r[0, 0][None, :], axis=1)


def _tc2_body(mean_r, h1_r, wl_r, wr_r, bl_r, cnt_r, w3_r, g_r, dinv_r):
  acc = jnp.broadcast_to(bl_r[...][None, :], (_NB, _H)).astype(jnp.float32)
  for k in range(4):
    acc = acc + jnp.dot(mean_r[k], wl_r[k], preferred_element_type=jnp.float32)
    acc = acc + jnp.dot(h1_r[k], wr_r[k], preferred_element_type=jnp.float32)
  h2 = jnp.maximum(acc, 0.0)
  dv = lax.rsqrt(cnt_r[...] + 1.0)
  g_r[...] = jnp.dot(h2, w3_r[...], preferred_element_type=jnp.float32) * dv
  dinv_r[...] = dv


def _tc3_body(p_r, dinv_r, b3_r, wc1_r, bc1_r, wc2_r, bc2_r, o_r):
  h3 = (p_r[0] + p_r[1]) * dinv_r[...] + b3_r[...][None, :]
  h4 = jnp.maximum(
      jnp.dot(h3, wc1_r[...], preferred_element_type=jnp.float32)
      + bc1_r[...][None, :], 0.0)
  o_r[...] = (jnp.dot(h4, wc2_r[...], preferred_element_type=jnp.float32)
              + bc2_r[...][None, :])


def kernel(x, edge_index, W1, att_src, att_dst, b1, Wl, bl, Wr, W3, b3,
           Wc1, bc1, Wc2, bc2):
  f32 = jnp.float32
  ei = edge_index.astype(jnp.int32)
  loop = jnp.arange(_N, dtype=jnp.int32)
  padF = _EPF - (_E + _N)
  padC = _EPC - _E
  srcF = jnp.concatenate([ei[0], loop, jnp.zeros((padF,), jnp.int32)])
  dstF = jnp.concatenate([ei[1], loop, jnp.full((padF,), _DUM, jnp.int32)])
  srcE = jnp.concatenate([ei[0], jnp.zeros((padC,), jnp.int32)])
  dstE = jnp.concatenate([ei[1], jnp.full((padC,), _DUM, jnp.int32)])
  xp = jnp.pad(x, ((0, _NP - _N), (0, 0)))
  W1r = W1.reshape(_H, 4, _H).transpose(1, 0, 2)
  Wlr = Wl.reshape(4, _H, _H)
  Wrr = Wr.reshape(4, _H, _H)

  hh, a_s3, a_d3 = pl.pallas_call(
      _tc1_body,
      grid=(4,),
      in_specs=[
          pl.BlockSpec((_NP, _H), lambda k: (0, 0)),
          pl.BlockSpec((1, _H, _H), lambda k: (k, 0, 0)),
          pl.BlockSpec((1, 1, _H), lambda k: (k, 0, 0)),
          pl.BlockSpec((1, 1, _H), lambda k: (k, 0, 0)),
      ],
      out_specs=[
          pl.BlockSpec((1, _NP, _H), lambda k: (k, 0, 0)),
          pl.BlockSpec((1, 1, _NP), lambda k: (k, 0, 0)),
          pl.BlockSpec((1, 1, _NP), lambda k: (k, 0, 0)),
      ],
      out_shape=[
          jax.ShapeDtypeStruct((4, _NP, _H), f32),
          jax.ShapeDtypeStruct((4, 1, _NP), f32),
          jax.ShapeDtypeStruct((4, 1, _NP), f32),
      ],
  )(xp, W1r, att_src.reshape(4, 1, _H), att_dst.reshape(4, 1, _H))

  hhf = hh.reshape(4 * _NP, _H)
  asf = a_s3.reshape(4 * _NP)
  adf = a_d3.reshape(4 * _NP)

  gat = functools.partial(
      pl.kernel,
      out_type=jax.ShapeDtypeStruct((4 * _NP, _H), f32),
      mesh=_mesh,
      compiler_params=pltpu.CompilerParams(needs_layout_passes=False),
      scratch_types=[
          pltpu.VMEM((_NP,), f32),
          pltpu.VMEM((_NP,), f32),
          pltpu.VMEM((80, _H), f32),
          pltpu.VMEM((80,), jnp.int32),
          pltpu.VMEM((_KG,), jnp.int32),
          pltpu.VMEM((_KG,), jnp.int32),
          pltpu.VMEM((_KG,), f32),
          pltpu.VMEM((_KG, _H), f32),
          pltpu.VMEM((_H,), f32),
          pltpu.VMEM_SHARED((_NP, _H), f32),
          pltpu.VMEM_SHARED((80, _H), f32),
          pltpu.SemaphoreType.DMA,
      ],
  )(_gat_sc)
  h1f = gat(hhf, asf, adf, b1, srcF, dstF)

  sage = functools.partial(
      pl.kernel,
      out_type=(jax.ShapeDtypeStruct((4 * _NP, _H), f32),
                jax.ShapeDtypeStruct((80, _H), f32)),
      mesh=_mesh,
      compiler_params=pltpu.CompilerParams(needs_layout_passes=False),
      scratch_types=[
          pltpu.VMEM((80, _H), f32),
          pltpu.VMEM((80,), jnp.int32),
          pltpu.VMEM((_K,), jnp.int32),
          pltpu.VMEM((_K,), jnp.int32),
          pltpu.VMEM((_K, _H), f32),
          pltpu.VMEM_SHARED((_NP, _H), f32),
          pltpu.VMEM_SHARED((80, _H), f32),
          pltpu.SemaphoreType.DMA,
      ],
  )(_sage_sc)
  meanf, cnt2 = sage(h1f, srcE, dstE)
  cnt = cnt2.reshape(_NP, 1)

  g, dinv = pl.pallas_call(
      _tc2_body,
      grid=(_NG,),
      in_specs=[
          pl.BlockSpec((4, _NB, _H), lambda i: (0, i, 0)),
          pl.BlockSpec((4, _NB, _H), lambda i: (0, i, 0)),
          pl.BlockSpec((4, _H, _H), lambda i: (0, 0, 0)),
          pl.BlockSpec((4, _H, _H), lambda i: (0, 0, 0)),
          pl.BlockSpec((_H,), lambda i: (0,)),
          pl.BlockSpec((_NB, 1), lambda i: (i, 0)),
          pl.BlockSpec((_H, _H), lambda i: (0, 0)),
      ],
      out_specs=[
          pl.BlockSpec((_NB, _H), lambda i: (i, 0)),
          pl.BlockSpec((_NB, 1), lambda i: (i, 0)),
      ],
      out_shape=[
          jax.ShapeDtypeStruct((_NP, _H), f32),
          jax.ShapeDtypeStruct((_NP, 1), f32),
      ],
  )(meanf.reshape(4, _NP, _H), h1f.reshape(4, _NP, _H), Wlr, Wrr, bl, cnt, W3)

  gcn = functools.partial(
      pl.kernel,
      out_type=jax.ShapeDtypeStruct((2 * _NP, _H), f32),
      mesh=_mesh,
      compiler_params=pltpu.CompilerParams(needs_layout_passes=False),
      scratch_types=[
          pltpu.VMEM((_K,), jnp.int32),
          pltpu.VMEM((_K,), jnp.int32),
          pltpu.VMEM((_K, _H), f32),
          pltpu.VMEM_SHARED((_NP, _H), f32),
          pltpu.SemaphoreType.DMA,
      ],
  )(_gcn_sc)
  partf = gcn(g, srcF, dstF)

  out = pl.pallas_call(
      _tc3_body,
      grid=(_NG,),
      in_specs=[
          pl.BlockSpec((2, _NB, _H), lambda i: (0, i, 0)),
          pl.BlockSpec((_NB, 1), lambda i: (i, 0)),
          pl.BlockSpec((_H,), lambda i: (0,)),
          pl.BlockSpec((_H, 64), lambda i: (0, 0)),
          pl.BlockSpec((64,), lambda i: (0,)),
          pl.BlockSpec((64, 16), lambda i: (0, 0)),
          pl.BlockSpec((16,), lambda i: (0,)),
      ],
      out_specs=pl.BlockSpec((_NB, 16), lambda i: (i, 0)),
      out_shape=jax.ShapeDtypeStruct((_NP, 16), f32),
  )(partf.reshape(2, _NP, _H), dinv, b3, Wc1, bc1, Wc2, bc2)

  return out[:_N]


# speedup vs baseline: 1.7268x; 1.0422x over previous
import functools

import jax
import jax.numpy as jnp
from jax import lax
from jax.experimental import pallas as pl
from jax.experimental.pallas import tpu as pltpu
from jax.experimental.pallas import tpu_sc as plsc

_N = 10000
_NP = 10240
_DUM = _N
_E = 320000
_EPF = 331776
_EPC = 321536
_K = 128
_RPT = _NP // 16
_H = 128
_NB = 2048
_NG = _NP // _NB

_mesh = plsc.VectorSubcoreMesh(core_axis_name="c", subcore_axis_name="s")


def _fill_identity(idqr):
  for g in range(5):
    idqr[pl.ds(g * 16, 16)] = lax.iota(jnp.int32, 16) + g * 16


def _zero_rows(buf, nrows):
  zv = jnp.zeros((16,), jnp.float32)

  def _zrow(i, _):
    for jj in range(8):
      buf[i, pl.ds(jj * 16, 16)] = zv
    return 0

  lax.fori_loop(0, nrows, _zrow, 0)


def _zero_shared_slice(wch, acc, s):
  _zero_rows(wch, 128)
  for ch in range(5):
    pltpu.sync_copy(wch, acc.at[pl.ds(s * _RPT + ch * 128, 128)])


_KG = 128


def _gat_sc(hh, a_s, a_d, b1f, srcF, dstF, h1o,
            asb, adb, denl, idqr, srcb, dstb, exb, rows, b1b,
            acc, den_sh, sem):
  c = lax.axis_index("c")
  s = lax.axis_index("s")
  ept = _EPF // 16
  _fill_identity(idqr)
  for j in range(2):
    head = c * 2 + j
    hoff = head * _NP
    pltpu.sync_copy(a_s.at[pl.ds(hoff, _NP)], asb)
    pltpu.sync_copy(a_d.at[pl.ds(hoff, _NP)], adb)
    pltpu.sync_copy(b1f.at[pl.ds(head * _H, _H)], b1b)

    _zero_rows(denl, 80)
    _zero_rows(rows, 64)
    for ch in range(10):
      pltpu.sync_copy(rows.at[pl.ds(0, 64)],
                      acc.at[pl.ds(s * _RPT + ch * 64, 64)])

    @pl.when(s < 10)
    def _():
      pltpu.sync_copy(denl.at[pl.ds(0, 8)], den_sh.at[pl.ds(s * 8, 8)])

    plsc.subcore_barrier()

    def _batch(b, _):
      e0 = s * ept + b * _KG
      pltpu.sync_copy(srcF.at[pl.ds(e0, _KG)], srcb)
      pltpu.sync_copy(dstF.at[pl.ds(e0, _KG)], dstb)
      for g in range(8):
        sl = pl.ds(g * 16, 16)
        s16 = srcb[sl]
        d16 = dstb[sl]
        asv = plsc.load_gather(asb, [s16])
        adv = plsc.load_gather(adb, [d16])
        al = asv + adv
        al = jnp.maximum(al, 0.2 * al)
        ex = jnp.exp(al)
        exb[sl] = ex
        plsc.addupdate_scatter(
            denl,
            [lax.shift_right_logical(d16, 7), lax.bitwise_and(d16, 127)], ex)
        srcb[sl] = s16 + hoff
      pltpu.async_copy(hh.at[srcb], rows, sem).wait()

      def _scale(k, _):
        wv = plsc.load_gather(exb, [jnp.full((16,), k, jnp.int32)])
        for jj in range(8):
          sl2 = pl.ds(jj * 16, 16)
          rows[k, sl2] = rows[k, sl2] * wv
        return 0

      lax.fori_loop(0, _KG, _scale, 0)
      pltpu.sync_copy(rows, acc.at[dstb], add=True)
      return 0

    lax.fori_loop(0, ept // _KG, _batch, 0)
    pltpu.sync_copy(denl, den_sh.at[idqr], add=True)
    plsc.subcore_barrier()
    pltpu.sync_copy(den_sh, denl)
    for ch in range(10):
      r0 = s * _RPT + ch * 64
      pltpu.sync_copy(acc.at[pl.ds(r0, 64)], rows.at[pl.ds(0, 64)])

      def _wrow(i, _):
        f = r0 + i
        dsp = plsc.load_gather(
            denl, [jnp.full((16,), lax.shift_right_logical(f, 7), jnp.int32),
                   jnp.full((16,), lax.bitwise_and(f, 127), jnp.int32)]
        ) + 1e-16
        for jj in range(8):
          sl2 = pl.ds(jj * 16, 16)
          rows[i, sl2] = jnp.maximum(rows[i, sl2] / dsp + b1b[sl2], 0.0)
        return 0

      lax.fori_loop(0, 64, _wrow, 0)
      pltpu.sync_copy(rows.at[pl.ds(0, 64)], h1o.at[pl.ds(hoff + r0, 64)])
    plsc.subcore_barrier()


def _sage_sc(h1f, srcE, dstE, meanf, cnto,
             cntl, idqr, srcb, dstb, rows, acc, cnt_sh, sem):
  c = lax.axis_index("c")
  s = lax.axis_index("s")
  ept = _EPC // 16
  _fill_identity(idqr)
  ones = jnp.full((16,), 1.0, jnp.float32)
  _zero_rows(cntl, 80)

  @pl.when(s < 10)
  def _():
    pltpu.sync_copy(cntl.at[pl.ds(0, 8)], cnt_sh.at[pl.ds(s * 8, 8)])
  for j in range(2):
    chunk = c * 2 + j
    coff = chunk * _NP
    _zero_shared_slice(rows, acc, s)
    plsc.subcore_barrier()

    def _batch(b, _):
      e0 = s * ept + b * _K
      pltpu.sync_copy(srcE.at[pl.ds(e0, _K)], srcb)
      pltpu.sync_copy(dstE.at[pl.ds(e0, _K)], dstb)
      for g in range(8):
        sl = pl.ds(g * 16, 16)
        s16 = srcb[sl]
        if j == 0:
          d16 = dstb[sl]
          plsc.addupdate_scatter(
              cntl,
              [lax.shift_right_logical(d16, 7), lax.bitwise_and(d16, 127)],
              ones)
        srcb[sl] = s16 + coff
      pltpu.async_copy(h1f.at[srcb], rows, sem).wait()
      pltpu.sync_copy(rows, acc.at[dstb], add=True)
      return 0

    lax.fori_loop(0, ept // _K, _batch, 0)
    if j == 0:
      pltpu.sync_copy(cntl, cnt_sh.at[idqr], add=True)
    plsc.subcore_barrier()
    if j == 0:
      pltpu.sync_copy(cnt_sh, cntl)

      @pl.when(jnp.logical_and(c == 0, s < 10))
      def _():
        pltpu.sync_copy(cntl.at[pl.ds(s * 8, 8)], cnto.at[pl.ds(s * 8, 8)])
    for ch in range(5):
      r0 = s * _RPT + ch * 128
      pltpu.sync_copy(acc.at[pl.ds(r0, 128)], rows)

      def _wrow(i, _):
        f = r0 + i
        csp = jnp.maximum(
            plsc.load_gather(
                cntl,
                [jnp.full((16,), lax.shift_right_logical(f, 7), jnp.int32),
                 jnp.full((16,), lax.bitwise_and(f, 127), jnp.int32)]), 1.0)
        for jj in range(8):
          sl2 = pl.ds(jj * 16, 16)
          rows[i, sl2] = rows[i, sl2] / csp
        return 0

      lax.fori_loop(0, 128, _wrow, 0)
      pltpu.sync_copy(rows, meanf.at[pl.ds(coff + r0, 128)])
    plsc.subcore_barrier()


def _gcn_sc(gf, srcF, dstF, partf, srcb, dstb, rows, acc, sem):
  wch = rows
  c = lax.axis_index("c")
  s = lax.axis_index("s")
  ept = _EPF // 32
  _zero_shared_slice(wch, acc, s)
  plsc.subcore_barrier()
  t = c * 16 + s

  def _batch(b, _):
    e0 = t * ept + b * _K
    pltpu.sync_copy(srcF.at[pl.ds(e0, _K)], srcb)
    pltpu.sync_copy(dstF.at[pl.ds(e0, _K)], dstb)
    pltpu.async_copy(gf.at[srcb], rows, sem).wait()
    pltpu.sync_copy(rows, acc.at[dstb], add=True)
    return 0

  lax.fori_loop(0, ept // _K, _batch, 0)
  plsc.subcore_barrier()
  for ch in range(5):
    r0 = s * _RPT + ch * 128
    pltpu.sync_copy(acc.at[pl.ds(r0, 128)], wch)
    pltpu.sync_copy(wch, partf.at[pl.ds(c * _NP + r0, 128)])


def _tc1_body(x_r, w1_r, asr_r, adr_r, hh_r, aso_r, ado_r):
  hk = jnp.dot(x_r[...], w1_r[0], preferred_element_type=jnp.float32)
  hh_r[0] = hk
  aso_r[0, 0] = jnp.sum(hk * asr_r[0, 0][None, :], axis=1)
  ado_r[0, 0] = jnp.sum(hk * adr_r[0, 0][None, :], axis=1)


def _tc2_body(mean_r, h1_r, wl_r, wr_r, bl_r, cnt_r, w3_r, g_r, dinv_r):
  acc = jnp.broadcast_to(bl_r[...][None, :], (_NB, _H)).astype(jnp.float32)
  for k in range(4):
    acc = acc + jnp.dot(mean_r[k], wl_r[k], preferred_element_type=jnp.float32)
    acc = acc + jnp.dot(h1_r[k], wr_r[k], preferred_element_type=jnp.float32)
  h2 = jnp.maximum(acc, 0.0)
  dv = lax.rsqrt(cnt_r[...] + 1.0)
  g_r[...] = jnp.dot(h2, w3_r[...], preferred_element_type=jnp.float32) * dv
  dinv_r[...] = dv


def _tc3_body(p_r, dinv_r, b3_r, wc1_r, bc1_r, wc2_r, bc2_r, o_r):
  h3 = (p_r[0] + p_r[1]) * dinv_r[...] + b3_r[...][None, :]
  h4 = jnp.maximum(
      jnp.dot(h3, wc1_r[...], preferred_element_type=jnp.float32)
      + bc1_r[...][None, :], 0.0)
  o_r[...] = (jnp.dot(h4, wc2_r[...], preferred_element_type=jnp.float32)
              + bc2_r[...][None, :])


def kernel(x, edge_index, W1, att_src, att_dst, b1, Wl, bl, Wr, W3, b3,
           Wc1, bc1, Wc2, bc2):
  f32 = jnp.float32
  ei = edge_index.astype(jnp.int32)
  loop = jnp.arange(_N, dtype=jnp.int32)
  padF = _EPF - (_E + _N)
  padC = _EPC - _E
  srcF = jnp.concatenate([ei[0], loop, jnp.zeros((padF,), jnp.int32)])
  dstF = jnp.concatenate([ei[1], loop, jnp.full((padF,), _DUM, jnp.int32)])
  srcE = jnp.concatenate([ei[0], jnp.zeros((padC,), jnp.int32)])
  dstE = jnp.concatenate([ei[1], jnp.full((padC,), _DUM, jnp.int32)])
  xp = jnp.pad(x, ((0, _NP - _N), (0, 0)))
  W1r = W1.reshape(_H, 4, _H).transpose(1, 0, 2)
  Wlr = Wl.reshape(4, _H, _H)
  Wrr = Wr.reshape(4, _H, _H)

  hh, a_s3, a_d3 = pl.pallas_call(
      _tc1_body,
      grid=(4,),
      in_specs=[
          pl.BlockSpec((_NP, _H), lambda k: (0, 0)),
          pl.BlockSpec((1, _H, _H), lambda k: (k, 0, 0)),
          pl.BlockSpec((1, 1, _H), lambda k: (k, 0, 0)),
          pl.BlockSpec((1, 1, _H), lambda k: (k, 0, 0)),
      ],
      out_specs=[
          pl.BlockSpec((1, _NP, _H), lambda k: (k, 0, 0)),
          pl.BlockSpec((1, 1, _NP), lambda k: (k, 0, 0)),
          pl.BlockSpec((1, 1, _NP), lambda k: (k, 0, 0)),
      ],
      out_shape=[
          jax.ShapeDtypeStruct((4, _NP, _H), f32),
          jax.ShapeDtypeStruct((4, 1, _NP), f32),
          jax.ShapeDtypeStruct((4, 1, _NP), f32),
      ],
  )(xp, W1r, att_src.reshape(4, 1, _H), att_dst.reshape(4, 1, _H))

  hhf = hh.reshape(4 * _NP, _H)
  asf = a_s3.reshape(4 * _NP)
  adf = a_d3.reshape(4 * _NP)

  gat = functools.partial(
      pl.kernel,
      out_type=jax.ShapeDtypeStruct((4 * _NP, _H), f32),
      mesh=_mesh,
      compiler_params=pltpu.CompilerParams(needs_layout_passes=False),
      scratch_types=[
          pltpu.VMEM((_NP,), f32),
          pltpu.VMEM((_NP,), f32),
          pltpu.VMEM((80, _H), f32),
          pltpu.VMEM((80,), jnp.int32),
          pltpu.VMEM((_KG,), jnp.int32),
          pltpu.VMEM((_KG,), jnp.int32),
          pltpu.VMEM((_KG,), f32),
          pltpu.VMEM((_KG, _H), f32),
          pltpu.VMEM((_H,), f32),
          pltpu.VMEM_SHARED((_NP, _H), f32),
          pltpu.VMEM_SHARED((80, _H), f32),
          pltpu.SemaphoreType.DMA,
      ],
  )(_gat_sc)
  h1f = gat(hhf, asf, adf, b1, srcF, dstF)

  sage = functools.partial(
      pl.kernel,
      out_type=(jax.ShapeDtypeStruct((4 * _NP, _H), f32),
                jax.ShapeDtypeStruct((80, _H), f32)),
      mesh=_mesh,
      compiler_params=pltpu.CompilerParams(needs_layout_passes=False),
      scratch_types=[
          pltpu.VMEM((80, _H), f32),
          pltpu.VMEM((80,), jnp.int32),
          pltpu.VMEM((_K,), jnp.int32),
          pltpu.VMEM((_K,), jnp.int32),
          pltpu.VMEM((_K, _H), f32),
          pltpu.VMEM_SHARED((_NP, _H), f32),
          pltpu.VMEM_SHARED((80, _H), f32),
          pltpu.SemaphoreType.DMA,
      ],
  )(_sage_sc)
  meanf, cnt2 = sage(h1f, srcE, dstE)
  cnt = cnt2.reshape(_NP, 1)

  g, dinv = pl.pallas_call(
      _tc2_body,
      grid=(_NG,),
      in_specs=[
          pl.BlockSpec((4, _NB, _H), lambda i: (0, i, 0)),
          pl.BlockSpec((4, _NB, _H), lambda i: (0, i, 0)),
          pl.BlockSpec((4, _H, _H), lambda i: (0, 0, 0)),
          pl.BlockSpec((4, _H, _H), lambda i: (0, 0, 0)),
          pl.BlockSpec((_H,), lambda i: (0,)),
          pl.BlockSpec((_NB, 1), lambda i: (i, 0)),
          pl.BlockSpec((_H, _H), lambda i: (0, 0)),
      ],
      out_specs=[
          pl.BlockSpec((_NB, _H), lambda i: (i, 0)),
          pl.BlockSpec((_NB, 1), lambda i: (i, 0)),
      ],
      out_shape=[
          jax.ShapeDtypeStruct((_NP, _H), f32),
          jax.ShapeDtypeStruct((_NP, 1), f32),
      ],
  )(meanf.reshape(4, _NP, _H), h1f.reshape(4, _NP, _H), Wlr, Wrr, bl, cnt, W3)

  gcn = functools.partial(
      pl.kernel,
      out_type=jax.ShapeDtypeStruct((2 * _NP, _H), f32),
      mesh=_mesh,
      compiler_params=pltpu.CompilerParams(needs_layout_passes=False),
      scratch_types=[
          pltpu.VMEM((_K,), jnp.int32),
          pltpu.VMEM((_K,), jnp.int32),
          pltpu.VMEM((_K, _H), f32),
          pltpu.VMEM_SHARED((_NP, _H), f32),
          pltpu.SemaphoreType.DMA,
      ],
  )(_gcn_sc)
  partf = gcn(g, srcF, dstF)

  out = pl.pallas_call(
      _tc3_body,
      grid=(_NG,),
      in_specs=[
          pl.BlockSpec((2, _NB, _H), lambda i: (0, i, 0)),
          pl.BlockSpec((_NB, 1), lambda i: (i, 0)),
          pl.BlockSpec((_H,), lambda i: (0,)),
          pl.BlockSpec((_H, 64), lambda i: (0, 0)),
          pl.BlockSpec((64,), lambda i: (0,)),
          pl.BlockSpec((64, 16), lambda i: (0, 0)),
          pl.BlockSpec((16,), lambda i: (0,)),
      ],
      out_specs=pl.BlockSpec((_NB, 16), lambda i: (i, 0)),
      out_shape=jax.ShapeDtypeStruct((_NP, 16), f32),
  )(partf.reshape(2, _NP, _H), dinv, b3, Wc1, bc1, Wc2, bc2)

  return out[:_N]


# speedup vs baseline: 1.7302x; 1.0019x over previous
import functools

import jax
import jax.numpy as jnp
from jax import lax
from jax.experimental import pallas as pl
from jax.experimental.pallas import tpu as pltpu
from jax.experimental.pallas import tpu_sc as plsc

_N = 10000
_NP = 10240
_DUM = _N
_E = 320000
_EPF = 331776
_EPC = 321536
_K = 128
_RPT = _NP // 16
_H = 128
_NB = 2048
_NG = _NP // _NB

_mesh = plsc.VectorSubcoreMesh(core_axis_name="c", subcore_axis_name="s")


def _fill_identity(idqr):
  for g in range(5):
    idqr[pl.ds(g * 16, 16)] = lax.iota(jnp.int32, 16) + g * 16


def _zero_rows(buf, nrows):
  zv = jnp.zeros((16,), jnp.float32)

  def _zrow(i, _):
    for jj in range(8):
      buf[i, pl.ds(jj * 16, 16)] = zv
    return 0

  lax.fori_loop(0, nrows, _zrow, 0)


def _zero_shared_slice(wch, acc, s):
  _zero_rows(wch, 128)
  for ch in range(5):
    pltpu.sync_copy(wch, acc.at[pl.ds(s * _RPT + ch * 128, 128)])


_KG = 128


def _gat_sc(hh, a_s, a_d, b1f, srcF, dstF, h1o,
            asb, adb, denl, idqr, srcb, dstb, exb, rows, b1b,
            acc, den_sh, sem):
  c = lax.axis_index("c")
  s = lax.axis_index("s")
  ept = _EPF // 16
  _fill_identity(idqr)
  for j in range(2):
    head = c * 2 + j
    hoff = head * _NP
    pltpu.sync_copy(a_s.at[pl.ds(hoff, _NP)], asb)
    pltpu.sync_copy(a_d.at[pl.ds(hoff, _NP)], adb)
    pltpu.sync_copy(b1f.at[pl.ds(head * _H, _H)], b1b)

    _zero_rows(denl, 80)
    _zero_shared_slice(rows, acc, s)

    @pl.when(s < 10)
    def _():
      pltpu.sync_copy(denl.at[pl.ds(0, 8)], den_sh.at[pl.ds(s * 8, 8)])

    plsc.subcore_barrier()

    def _batch(b, _):
      e0 = s * ept + b * _KG
      pltpu.sync_copy(srcF.at[pl.ds(e0, _KG)], srcb)
      pltpu.sync_copy(dstF.at[pl.ds(e0, _KG)], dstb)
      for g in range(8):
        sl = pl.ds(g * 16, 16)
        s16 = srcb[sl]
        d16 = dstb[sl]
        asv = plsc.load_gather(asb, [s16])
        adv = plsc.load_gather(adb, [d16])
        al = asv + adv
        al = jnp.maximum(al, 0.2 * al)
        ex = jnp.exp(al)
        exb[sl] = ex
        plsc.addupdate_scatter(
            denl,
            [lax.shift_right_logical(d16, 7), lax.bitwise_and(d16, 127)], ex)
        srcb[sl] = s16 + hoff
      pltpu.async_copy(hh.at[srcb], rows, sem).wait()

      def _scale(k, _):
        wv = plsc.load_gather(exb, [jnp.full((16,), k, jnp.int32)])
        for jj in range(8):
          sl2 = pl.ds(jj * 16, 16)
          rows[k, sl2] = rows[k, sl2] * wv
        return 0

      lax.fori_loop(0, _KG, _scale, 0)
      pltpu.sync_copy(rows, acc.at[dstb], add=True)
      return 0

    lax.fori_loop(0, ept // _KG, _batch, 0)
    pltpu.sync_copy(denl, den_sh.at[idqr], add=True)
    plsc.subcore_barrier()
    pltpu.sync_copy(den_sh, denl)
    for ch in range(5):
      r0 = s * _RPT + ch * 128
      pltpu.sync_copy(acc.at[pl.ds(r0, 128)], rows)

      def _wrow(i, _):
        f = r0 + i
        dsp = plsc.load_gather(
            denl, [jnp.full((16,), lax.shift_right_logical(f, 7), jnp.int32),
                   jnp.full((16,), lax.bitwise_and(f, 127), jnp.int32)]
        ) + 1e-16
        for jj in range(8):
          sl2 = pl.ds(jj * 16, 16)
          rows[i, sl2] = jnp.maximum(rows[i, sl2] / dsp + b1b[sl2], 0.0)
        return 0

      lax.fori_loop(0, 128, _wrow, 0)
      pltpu.sync_copy(rows, h1o.at[pl.ds(hoff + r0, 128)])
    plsc.subcore_barrier()


def _sage_sc(h1f, srcE, dstE, meanf, cnto,
             cntl, idqr, srcb, dstb, rows, acc, cnt_sh, sem):
  c = lax.axis_index("c")
  s = lax.axis_index("s")
  ept = _EPC // 16
  _fill_identity(idqr)
  ones = jnp.full((16,), 1.0, jnp.float32)
  _zero_rows(cntl, 80)

  @pl.when(s < 10)
  def _():
    pltpu.sync_copy(cntl.at[pl.ds(0, 8)], cnt_sh.at[pl.ds(s * 8, 8)])
  for j in range(2):
    chunk = c * 2 + j
    coff = chunk * _NP
    _zero_shared_slice(rows, acc, s)
    plsc.subcore_barrier()

    def _batch(b, _):
      e0 = s * ept + b * _K
      pltpu.sync_copy(srcE.at[pl.ds(e0, _K)], srcb)
      pltpu.sync_copy(dstE.at[pl.ds(e0, _K)], dstb)
      for g in range(8):
        sl = pl.ds(g * 16, 16)
        s16 = srcb[sl]
        if j == 0:
          d16 = dstb[sl]
          plsc.addupdate_scatter(
              cntl,
              [lax.shift_right_logical(d16, 7), lax.bitwise_and(d16, 127)],
              ones)
        srcb[sl] = s16 + coff
      pltpu.async_copy(h1f.at[srcb], rows, sem).wait()
      pltpu.sync_copy(rows, acc.at[dstb], add=True)
      return 0

    lax.fori_loop(0, ept // _K, _batch, 0)
    if j == 0:
      pltpu.sync_copy(cntl, cnt_sh.at[idqr], add=True)
    plsc.subcore_barrier()
    if j == 0:
      pltpu.sync_copy(cnt_sh, cntl)

      @pl.when(jnp.logical_and(c == 0, s < 10))
      def _():
        pltpu.sync_copy(cntl.at[pl.ds(s * 8, 8)], cnto.at[pl.ds(s * 8, 8)])
    for ch in range(5):
      r0 = s * _RPT + ch * 128
      pltpu.sync_copy(acc.at[pl.ds(r0, 128)], rows)

      def _wrow(i, _):
        f = r0 + i
        csp = jnp.maximum(
            plsc.load_gather(
                cntl,
                [jnp.full((16,), lax.shift_right_logical(f, 7), jnp.int32),
                 jnp.full((16,), lax.bitwise_and(f, 127), jnp.int32)]), 1.0)
        for jj in range(8):
          sl2 = pl.ds(jj * 16, 16)
          rows[i, sl2] = rows[i, sl2] / csp
        return 0

      lax.fori_loop(0, 128, _wrow, 0)
      pltpu.sync_copy(rows, meanf.at[pl.ds(coff + r0, 128)])
    plsc.subcore_barrier()


def _gcn_sc(gf, srcF, dstF, partf, srcb, dstb, rows, acc, sem):
  wch = rows
  c = lax.axis_index("c")
  s = lax.axis_index("s")
  ept = _EPF // 32
  _zero_shared_slice(wch, acc, s)
  plsc.subcore_barrier()
  t = c * 16 + s

  def _batch(b, _):
    e0 = t * ept + b * _K
    pltpu.sync_copy(srcF.at[pl.ds(e0, _K)], srcb)
    pltpu.sync_copy(dstF.at[pl.ds(e0, _K)], dstb)
    pltpu.async_copy(gf.at[srcb], rows, sem).wait()
    pltpu.sync_copy(rows, acc.at[dstb], add=True)
    return 0

  lax.fori_loop(0, ept // _K, _batch, 0)
  plsc.subcore_barrier()
  for ch in range(5):
    r0 = s * _RPT + ch * 128
    pltpu.sync_copy(acc.at[pl.ds(r0, 128)], wch)
    pltpu.sync_copy(wch, partf.at[pl.ds(c * _NP + r0, 128)])


def _tc1_body(x_r, w1_r, asr_r, adr_r, hh_r, aso_r, ado_r):
  hk = jnp.dot(x_r[...], w1_r[0], preferred_element_type=jnp.float32)
  hh_r[0] = hk
  aso_r[0, 0] = jnp.sum(hk * asr_r[0, 0][None, :], axis=1)
  ado_r[0, 0] = jnp.sum(hk * adr_r[0, 0][None, :], axis=1)


def _tc2_body(mean_r, h1_r, wl_r, wr_r, bl_r, cnt_r, w3_r, g_r, dinv_r):
  acc = jnp.broadcast_to(bl_r[...][None, :], (_NB, _H)).astype(jnp.float32)
  for k in range(4):
    acc = acc + jnp.dot(mean_r[k], wl_r[k], preferred_element_type=jnp.float32)
    acc = acc + jnp.dot(h1_r[k], wr_r[k], preferred_element_type=jnp.float32)
  h2 = jnp.maximum(acc, 0.0)
  dv = lax.rsqrt(cnt_r[...] + 1.0)
  g_r[...] = jnp.dot(h2, w3_r[...], preferred_element_type=jnp.float32) * dv
  dinv_r[...] = dv


def _tc3_body(p_r, dinv_r, b3_r, wc1_r, bc1_r, wc2_r, bc2_r, o_r):
  h3 = (p_r[0] + p_r[1]) * dinv_r[...] + b3_r[...][None, :]
  h4 = jnp.maximum(
      jnp.dot(h3, wc1_r[...], preferred_element_type=jnp.float32)
      + bc1_r[...][None, :], 0.0)
  o_r[...] = (jnp.dot(h4, wc2_r[...], preferred_element_type=jnp.float32)
              + bc2_r[...][None, :])


def kernel(x, edge_index, W1, att_src, att_dst, b1, Wl, bl, Wr, W3, b3,
           Wc1, bc1, Wc2, bc2):
  f32 = jnp.float32
  ei = edge_index.astype(jnp.int32)
  loop = jnp.arange(_N, dtype=jnp.int32)
  padF = _EPF - (_E + _N)
  padC = _EPC - _E
  srcF = jnp.concatenate([ei[0], loop, jnp.zeros((padF,), jnp.int32)])
  dstF = jnp.concatenate([ei[1], loop, jnp.full((padF,), _DUM, jnp.int32)])
  srcE = jnp.concatenate([ei[0], jnp.zeros((padC,), jnp.int32)])
  dstE = jnp.concatenate([ei[1], jnp.full((padC,), _DUM, jnp.int32)])
  xp = jnp.pad(x, ((0, _NP - _N), (0, 0)))
  W1r = W1.reshape(_H, 4, _H).transpose(1, 0, 2)
  Wlr = Wl.reshape(4, _H, _H)
  Wrr = Wr.reshape(4, _H, _H)

  hh, a_s3, a_d3 = pl.pallas_call(
      _tc1_body,
      grid=(4,),
      in_specs=[
          pl.BlockSpec((_NP, _H), lambda k: (0, 0)),
          pl.BlockSpec((1, _H, _H), lambda k: (k, 0, 0)),
          pl.BlockSpec((1, 1, _H), lambda k: (k, 0, 0)),
          pl.BlockSpec((1, 1, _H), lambda k: (k, 0, 0)),
      ],
      out_specs=[
          pl.BlockSpec((1, _NP, _H), lambda k: (k, 0, 0)),
          pl.BlockSpec((1, 1, _NP), lambda k: (k, 0, 0)),
          pl.BlockSpec((1, 1, _NP), lambda k: (k, 0, 0)),
      ],
      out_shape=[
          jax.ShapeDtypeStruct((4, _NP, _H), f32),
          jax.ShapeDtypeStruct((4, 1, _NP), f32),
          jax.ShapeDtypeStruct((4, 1, _NP), f32),
      ],
  )(xp, W1r, att_src.reshape(4, 1, _H), att_dst.reshape(4, 1, _H))

  hhf = hh.reshape(4 * _NP, _H)
  asf = a_s3.reshape(4 * _NP)
  adf = a_d3.reshape(4 * _NP)

  gat = functools.partial(
      pl.kernel,
      out_type=jax.ShapeDtypeStruct((4 * _NP, _H), f32),
      mesh=_mesh,
      compiler_params=pltpu.CompilerParams(needs_layout_passes=False),
      scratch_types=[
          pltpu.VMEM((_NP,), f32),
          pltpu.VMEM((_NP,), f32),
          pltpu.VMEM((80, _H), f32),
          pltpu.VMEM((80,), jnp.int32),
          pltpu.VMEM((_KG,), jnp.int32),
          pltpu.VMEM((_KG,), jnp.int32),
          pltpu.VMEM((_KG,), f32),
          pltpu.VMEM((_KG, _H), f32),
          pltpu.VMEM((_H,), f32),
          pltpu.VMEM_SHARED((_NP, _H), f32),
          pltpu.VMEM_SHARED((80, _H), f32),
          pltpu.SemaphoreType.DMA,
      ],
  )(_gat_sc)
  h1f = gat(hhf, asf, adf, b1, srcF, dstF)

  sage = functools.partial(
      pl.kernel,
      out_type=(jax.ShapeDtypeStruct((4 * _NP, _H), f32),
                jax.ShapeDtypeStruct((80, _H), f32)),
      mesh=_mesh,
      compiler_params=pltpu.CompilerParams(needs_layout_passes=False),
      scratch_types=[
          pltpu.VMEM((80, _H), f32),
          pltpu.VMEM((80,), jnp.int32),
          pltpu.VMEM((_K,), jnp.int32),
          pltpu.VMEM((_K,), jnp.int32),
          pltpu.VMEM((_K, _H), f32),
          pltpu.VMEM_SHARED((_NP, _H), f32),
          pltpu.VMEM_SHARED((80, _H), f32),
          pltpu.SemaphoreType.DMA,
      ],
  )(_sage_sc)
  meanf, cnt2 = sage(h1f, srcE, dstE)
  cnt = cnt2.reshape(_NP, 1)

  g, dinv = pl.pallas_call(
      _tc2_body,
      grid=(_NG,),
      in_specs=[
          pl.BlockSpec((4, _NB, _H), lambda i: (0, i, 0)),
          pl.BlockSpec((4, _NB, _H), lambda i: (0, i, 0)),
          pl.BlockSpec((4, _H, _H), lambda i: (0, 0, 0)),
          pl.BlockSpec((4, _H, _H), lambda i: (0, 0, 0)),
          pl.BlockSpec((_H,), lambda i: (0,)),
          pl.BlockSpec((_NB, 1), lambda i: (i, 0)),
          pl.BlockSpec((_H, _H), lambda i: (0, 0)),
      ],
      out_specs=[
          pl.BlockSpec((_NB, _H), lambda i: (i, 0)),
          pl.BlockSpec((_NB, 1), lambda i: (i, 0)),
      ],
      out_shape=[
          jax.ShapeDtypeStruct((_NP, _H), f32),
          jax.ShapeDtypeStruct((_NP, 1), f32),
      ],
  )(meanf.reshape(4, _NP, _H), h1f.reshape(4, _NP, _H), Wlr, Wrr, bl, cnt, W3)

  gcn = functools.partial(
      pl.kernel,
      out_type=jax.ShapeDtypeStruct((2 * _NP, _H), f32),
      mesh=_mesh,
      compiler_params=pltpu.CompilerParams(needs_layout_passes=False),
      scratch_types=[
          pltpu.VMEM((_K,), jnp.int32),
          pltpu.VMEM((_K,), jnp.int32),
          pltpu.VMEM((_K, _H), f32),
          pltpu.VMEM_SHARED((_NP, _H), f32),
          pltpu.SemaphoreType.DMA,
      ],
  )(_gcn_sc)
  partf = gcn(g, srcF, dstF)

  out = pl.pallas_call(
      _tc3_body,
      grid=(_NG,),
      in_specs=[
          pl.BlockSpec((2, _NB, _H), lambda i: (0, i, 0)),
          pl.BlockSpec((_NB, 1), lambda i: (i, 0)),
          pl.BlockSpec((_H,), lambda i: (0,)),
          pl.BlockSpec((_H, 64), lambda i: (0, 0)),
          pl.BlockSpec((64,), lambda i: (0,)),
          pl.BlockSpec((64, 16), lambda i: (0, 0)),
          pl.BlockSpec((16,), lambda i: (0,)),
      ],
      out_specs=pl.BlockSpec((_NB, 16), lambda i: (i, 0)),
      out_shape=jax.ShapeDtypeStruct((_NP, 16), f32),
  )(partf.reshape(2, _NP, _H), dinv, b3, Wc1, bc1, Wc2, bc2)

  return out[:_N]
